# baseline (speedup 1.0000x reference)
import functools

import jax
import jax.numpy as jnp
from jax import lax
from jax.experimental import pallas as pl
from jax.experimental.pallas import tpu as pltpu
from jax.experimental.pallas import tpu_sc as plsc

SCALE = 32
NG = 2 * SCALE
NG3 = NG * NG * NG

_PAIR_OFFS = (0, NG, NG * NG, NG * NG + NG)

L = 16
CHUNK = 4096
NBUF = 2

_PACK_COLS = 1024
_PACK_ROWS = 512


def _pack_body(g_ref, o_ref):
    x = g_ref[...]
    xs = jnp.concatenate([x[1:], x[:1]])
    b = lax.bitcast_convert_type(x, jnp.uint32)
    bs = lax.bitcast_convert_type(xs, jnp.uint32)
    o_ref[...] = (bs & jnp.uint32(0xFFFF0000)) | (b >> 16)


def _pack_pairs(gflat):
    n = gflat.shape[0]
    blk = _PACK_ROWS * _PACK_COLS
    return pl.pallas_call(
        _pack_body,
        grid=(n // blk,),
        in_specs=[pl.BlockSpec((blk,), lambda i: (i,))],
        out_specs=pl.BlockSpec((blk,), lambda i: (i,)),
        out_shape=jax.ShapeDtypeStruct((n,), jnp.uint32),
    )(gflat)


def _make_sampler(B, N):
    NW = 32
    assert B == NW and N % (CHUNK * NBUF) == 0
    nch = N // CHUNK
    mesh = plsc.VectorSubcoreMesh(core_axis_name="c", subcore_axis_name="s")

    @functools.partial(
        pl.kernel,
        mesh=mesh,
        out_type=jax.ShapeDtypeStruct((B * N,), jnp.float32),
        scratch_types=(
            [pltpu.VMEM((3, CHUNK), jnp.float32) for _ in range(NBUF)]
            + [pltpu.VMEM((4 * CHUNK,), jnp.int32) for _ in range(NBUF)]
            + [pltpu.VMEM((4 * CHUNK,), jnp.uint32) for _ in range(NBUF)]
            + [pltpu.VMEM((CHUNK,), jnp.float32) for _ in range(NBUF)]
            + [pltpu.SemaphoreType.DMA for _ in range(3 * NBUF)]
        ),
    )
    def sampler(tb_hbm, pts_hbm, out_hbm, *bufs):
        xyz = bufs[0:2]
        idx = bufs[2:4]
        val = bufs[4:6]
        outv = bufs[6:8]
        sem_xyz = bufs[8:10]
        sem_g = bufs[10:12]
        sem_o = bufs[12:14]

        wid = lax.axis_index("s") * 2 + lax.axis_index("c")
        pbase = wid * N
        gbase = wid * NG3

        def start_xyz(ci, b):
            pltpu.async_copy(pts_hbm.at[:, pl.ds(pbase + ci * CHUNK, CHUNK)],
                             xyz[b], sem_xyz[b])

        def pass1(b):
            def grp(i, _):
                o = i * L
                xs = xyz[b][0, pl.ds(o, L)] + float(SCALE)
                ys = xyz[b][1, pl.ds(o, L)] + float(SCALE)
                zs = xyz[b][2, pl.ds(o, L)] + float(SCALE)
                fi = xs.astype(jnp.int32)
                fj = ys.astype(jnp.int32)
                fk = zs.astype(jnp.int32)
                ii = jnp.minimum(fi, NG - 2)
                jj = jnp.minimum(fj, NG - 2)
                kk = jnp.minimum(fk, NG - 2)
                lin0 = (ii << 12) + (jj << 6) + kk + gbase
                for t in range(4):
                    idx[b][pl.ds(t * CHUNK + o, L)] = lin0 + _PAIR_OFFS[t]

            lax.fori_loop(0, CHUNK // L, grp, None, unroll=4)

        def pass2(b):
            def grp(i, _):
                o = i * L
                xs = xyz[b][0, pl.ds(o, L)] + float(SCALE)
                ys = xyz[b][1, pl.ds(o, L)] + float(SCALE)
                zs = xyz[b][2, pl.ds(o, L)] + float(SCALE)
                tx = xs - xs.astype(jnp.int32).astype(jnp.float32)
                ty = ys - ys.astype(jnp.int32).astype(jnp.float32)
                tz = zs - zs.astype(jnp.int32).astype(jnp.float32)
                c = []
                for t in range(4):
                    pv = val[b][pl.ds(t * CHUNK + o, L)]
                    v0 = lax.bitcast_convert_type(pv << 16, jnp.float32)
                    v1 = lax.bitcast_convert_type(
                        pv & jnp.uint32(0xFFFF0000), jnp.float32)
                    c.append(v0 + tz * (v1 - v0))
                d0 = c[0] + ty * (c[1] - c[0])
                d1 = c[2] + ty * (c[3] - c[2])
                outv[b][pl.ds(o, L)] = d0 + tx * (d1 - d0)

            lax.fori_loop(0, CHUNK // L, grp, None, unroll=4)

        def start_gather(b):
            pltpu.async_copy(tb_hbm.at[idx[b]], val[b], sem_g[b])

        def start_out(ci, b):
            pltpu.async_copy(outv[b], out_hbm.at[pl.ds(pbase + ci * CHUNK, CHUNK)],
                             sem_o[b])

        def wait_xyz(b):
            pltpu.make_async_copy(pts_hbm.at[:, pl.ds(0, CHUNK)], xyz[b],
                                  sem_xyz[b]).wait()

        def wait_gather(b):
            pltpu.make_async_copy(tb_hbm.at[idx[b]], val[b], sem_g[b]).wait()

        def wait_out(b):
            pltpu.make_async_copy(outv[b], out_hbm.at[pl.ds(0, CHUNK)],
                                  sem_o[b]).wait()

        start_xyz(0, 0)
        start_xyz(1, 1)
        wait_xyz(0)
        pass1(0)
        start_gather(0)

        def step(ci, cur):
            nxt = 1 - cur

            @pl.when(ci + 1 < nch)
            def _():
                wait_xyz(nxt)
                pass1(nxt)
                start_gather(nxt)

            @pl.when(ci >= 2)
            def _():
                wait_out(cur)

            wait_gather(cur)
            pass2(cur)
            start_out(ci, cur)

            @pl.when(ci + 2 < nch)
            def _():
                start_xyz(ci + 2, cur)

        def two_steps(m, _):
            step(2 * m, 0)
            step(2 * m + 1, 1)
            return None

        lax.fori_loop(0, nch // 2, two_steps, None)
        wait_out(0)
        wait_out(1)

    return sampler


def kernel(grid, ptcloud):
    B, N = ptcloud.shape[0], ptcloud.shape[1]
    tb = _pack_pairs(grid.reshape(-1))
    pts = ptcloud.reshape(B * N, 3).T
    out = _make_sampler(B, N)(tb, pts)
    return out.reshape(B, N)


# speedup vs baseline: 1.0201x; 1.0201x over previous
import functools

import jax
import jax.numpy as jnp
from jax import lax
from jax.experimental import pallas as pl
from jax.experimental.pallas import tpu as pltpu
from jax.experimental.pallas import tpu_sc as plsc

SCALE = 32
NG = 2 * SCALE
NG3 = NG * NG * NG

_PAIR_OFFS = (0, NG, NG * NG, NG * NG + NG)

L = 16
CHUNK = 2048
NBUF = 2

_PACK_COLS = 1024
_PACK_ROWS = 512


def _pack_body(g_ref, o_ref):
    x = g_ref[...]
    xs = jnp.concatenate([x[1:], x[:1]])
    b = lax.bitcast_convert_type(x, jnp.uint32)
    bs = lax.bitcast_convert_type(xs, jnp.uint32)
    o_ref[...] = (bs & jnp.uint32(0xFFFF0000)) | (b >> 16)


def _pack_pairs(gflat):
    n = gflat.shape[0]
    blk = _PACK_ROWS * _PACK_COLS
    return pl.pallas_call(
        _pack_body,
        grid=(n // blk,),
        in_specs=[pl.BlockSpec((blk,), lambda i: (i,))],
        out_specs=pl.BlockSpec((blk,), lambda i: (i,)),
        out_shape=jax.ShapeDtypeStruct((n,), jnp.uint32),
    )(gflat)


def _make_sampler(B, N):
    NW = 32
    assert B == NW and N % (CHUNK * NBUF) == 0
    nch = N // CHUNK
    mesh = plsc.VectorSubcoreMesh(core_axis_name="c", subcore_axis_name="s")

    @functools.partial(
        pl.kernel,
        mesh=mesh,
        out_type=jax.ShapeDtypeStruct((B * N,), jnp.float32),
        scratch_types=(
            [pltpu.VMEM((3, CHUNK), jnp.float32) for _ in range(NBUF)]
            + [pltpu.VMEM((3, CHUNK), jnp.float32) for _ in range(NBUF)]
            + [pltpu.VMEM((4 * CHUNK,), jnp.int32) for _ in range(NBUF)]
            + [pltpu.VMEM((4 * CHUNK,), jnp.uint32) for _ in range(NBUF)]
            + [pltpu.VMEM((CHUNK,), jnp.float32) for _ in range(NBUF)]
            + [pltpu.SemaphoreType.DMA for _ in range(3 * NBUF)]
        ),
    )
    def sampler(tb_hbm, pts_hbm, out_hbm, *bufs):
        xyz = bufs[0:2]
        tbuf = bufs[2:4]
        idx = bufs[4:6]
        val = bufs[6:8]
        outv = bufs[8:10]
        sem_xyz = bufs[10:12]
        sem_g = bufs[12:14]
        sem_o = bufs[14:16]

        wid = lax.axis_index("s") * 2 + lax.axis_index("c")
        pbase = wid * N
        gbase = wid * NG3

        def start_xyz(ci, b):
            pltpu.async_copy(pts_hbm.at[:, pl.ds(pbase + ci * CHUNK, CHUNK)],
                             xyz[b], sem_xyz[b])

        def pass1(b):
            def grp(i, _):
                o = i * L
                xs = xyz[b][0, pl.ds(o, L)] + float(SCALE)
                ys = xyz[b][1, pl.ds(o, L)] + float(SCALE)
                zs = xyz[b][2, pl.ds(o, L)] + float(SCALE)
                fi = xs.astype(jnp.int32)
                fj = ys.astype(jnp.int32)
                fk = zs.astype(jnp.int32)
                tbuf[b][0, pl.ds(o, L)] = xs - fi.astype(jnp.float32)
                tbuf[b][1, pl.ds(o, L)] = ys - fj.astype(jnp.float32)
                tbuf[b][2, pl.ds(o, L)] = zs - fk.astype(jnp.float32)
                ii = jnp.minimum(fi, NG - 2)
                jj = jnp.minimum(fj, NG - 2)
                kk = jnp.minimum(fk, NG - 2)
                lin0 = (ii << 12) + (jj << 6) + kk + gbase
                for t in range(4):
                    idx[b][pl.ds(t * CHUNK + o, L)] = lin0 + _PAIR_OFFS[t]

            lax.fori_loop(0, CHUNK // L, grp, None, unroll=4)

        def pass2(b):
            def grp(i, _):
                o = i * L
                tx = tbuf[b][0, pl.ds(o, L)]
                ty = tbuf[b][1, pl.ds(o, L)]
                tz = tbuf[b][2, pl.ds(o, L)]
                c = []
                for t in range(4):
                    pv = val[b][pl.ds(t * CHUNK + o, L)]
                    v0 = lax.bitcast_convert_type(pv << 16, jnp.float32)
                    v1 = lax.bitcast_convert_type(
                        pv & jnp.uint32(0xFFFF0000), jnp.float32)
                    c.append(v0 + tz * (v1 - v0))
                d0 = c[0] + ty * (c[1] - c[0])
                d1 = c[2] + ty * (c[3] - c[2])
                outv[b][pl.ds(o, L)] = d0 + tx * (d1 - d0)

            lax.fori_loop(0, CHUNK // L, grp, None, unroll=4)

        def start_gather(b):
            pltpu.async_copy(tb_hbm.at[idx[b]], val[b], sem_g[b])

        def start_out(ci, b):
            pltpu.async_copy(outv[b], out_hbm.at[pl.ds(pbase + ci * CHUNK, CHUNK)],
                             sem_o[b])

        def wait_xyz(b):
            pltpu.make_async_copy(pts_hbm.at[:, pl.ds(0, CHUNK)], xyz[b],
                                  sem_xyz[b]).wait()

        def wait_gather(b):
            pltpu.make_async_copy(tb_hbm.at[idx[b]], val[b], sem_g[b]).wait()

        def wait_out(b):
            pltpu.make_async_copy(outv[b], out_hbm.at[pl.ds(0, CHUNK)],
                                  sem_o[b]).wait()

        start_xyz(0, 0)
        start_xyz(1, 1)
        wait_xyz(0)
        pass1(0)
        start_gather(0)

        def step(ci, cur):
            nxt = 1 - cur

            @pl.when(ci + 1 < nch)
            def _():
                wait_xyz(nxt)
                pass1(nxt)
                start_gather(nxt)

            @pl.when(ci + 2 < nch)
            def _():
                start_xyz(ci + 2, cur)

            @pl.when(ci >= 2)
            def _():
                wait_out(cur)

            wait_gather(cur)
            pass2(cur)
            start_out(ci, cur)

        def two_steps(m, _):
            step(2 * m, 0)
            step(2 * m + 1, 1)
            return None

        lax.fori_loop(0, nch // 2, two_steps, None)
        wait_out(0)
        wait_out(1)

    return sampler


def kernel(grid, ptcloud):
    B, N = ptcloud.shape[0], ptcloud.shape[1]
    tb = _pack_pairs(grid.reshape(-1))
    pts = ptcloud.reshape(B * N, 3).T
    out = _make_sampler(B, N)(tb, pts)
    return out.reshape(B, N)
